# initial kernel scaffold (unmeasured)
import jax
import jax.numpy as jnp
from jax import lax
from jax.experimental import pallas as pl
from jax.experimental.pallas import tpu as pltpu


def kernel(
    x,
):
    def body(*refs):
        pass

    out_shape = jax.ShapeDtypeStruct(..., jnp.float32)
    return pl.pallas_call(body, out_shape=out_shape)(...)


# baseline (device time: 6521 ns/iter reference)
import jax
import jax.numpy as jnp
from jax import lax
from jax.experimental import pallas as pl
from jax.experimental.pallas import tpu as pltpu

N_DEV = 4


def kernel(x):
    m_per, n = x.shape

    def body(x_ref, out_ref, send_buf, halo_buf, send_sems, recv_sems):
        my = lax.axis_index("i")
        left = lax.rem(my + N_DEV - 1, N_DEV)
        right = lax.rem(my + 1, N_DEV)

        send_buf[0, :, :] = x_ref[0:1, :]
        send_buf[1, :, :] = x_ref[m_per - 1 : m_per, :]

        barrier_sem = pltpu.get_barrier_semaphore()
        for nbr in (left, right):
            pl.semaphore_signal(
                barrier_sem,
                inc=1,
                device_id=(nbr,),
                device_id_type=pl.DeviceIdType.MESH,
            )
        pl.semaphore_wait(barrier_sem, 2)

        rdma_to_right = pltpu.make_async_remote_copy(
            src_ref=send_buf.at[1],
            dst_ref=halo_buf.at[0],
            send_sem=send_sems.at[0],
            recv_sem=recv_sems.at[0],
            device_id=(right,),
            device_id_type=pl.DeviceIdType.MESH,
        )
        rdma_to_left = pltpu.make_async_remote_copy(
            src_ref=send_buf.at[0],
            dst_ref=halo_buf.at[1],
            send_sem=send_sems.at[1],
            recv_sem=recv_sems.at[1],
            device_id=(left,),
            device_id_type=pl.DeviceIdType.MESH,
        )
        rdma_to_right.start()
        rdma_to_left.start()

        xv = x_ref[:, :]
        out_ref[1 : m_per - 1, :] = (
            0.25 * xv[0 : m_per - 2, :]
            + 0.5 * xv[1 : m_per - 1, :]
            + 0.25 * xv[2:m_per, :]
        )

        rdma_to_right.wait()
        halo_left = halo_buf[0, :, :]
        top = 0.25 * halo_left + 0.5 * xv[0:1, :] + 0.25 * xv[1:2, :]
        out_ref[0:1, :] = jnp.where(my == 0, xv[0:1, :], top)

        rdma_to_left.wait()
        halo_right = halo_buf[1, :, :]
        bot = (
            0.25 * xv[m_per - 2 : m_per - 1, :]
            + 0.5 * xv[m_per - 1 : m_per, :]
            + 0.25 * halo_right
        )
        out_ref[m_per - 1 : m_per, :] = jnp.where(
            my == N_DEV - 1, xv[m_per - 1 : m_per, :], bot
        )

    return pl.pallas_call(
        body,
        out_shape=jax.ShapeDtypeStruct((m_per, n), x.dtype),
        in_specs=[pl.BlockSpec(memory_space=pltpu.VMEM)],
        out_specs=pl.BlockSpec(memory_space=pltpu.VMEM),
        scratch_shapes=[
            pltpu.VMEM((2, 1, n), x.dtype),
            pltpu.VMEM((2, 1, n), x.dtype),
            pltpu.SemaphoreType.DMA((2,)),
            pltpu.SemaphoreType.DMA((2,)),
        ],
        compiler_params=pltpu.CompilerParams(collective_id=0),
    )(x)


# device time: 6504 ns/iter; 1.0026x vs baseline; 1.0026x over previous
import jax
import jax.numpy as jnp
from jax import lax
from jax.experimental import pallas as pl
from jax.experimental.pallas import tpu as pltpu

N_DEV = 4


def kernel(x):
    m_per, n = x.shape

    def body(x_ref, out_ref, halo_buf, send_sems, recv_sems):
        my = lax.axis_index("i")
        left = lax.rem(my + N_DEV - 1, N_DEV)
        right = lax.rem(my + 1, N_DEV)

        barrier_sem = pltpu.get_barrier_semaphore()
        for nbr in (left, right):
            pl.semaphore_signal(
                barrier_sem,
                inc=1,
                device_id=(nbr,),
                device_id_type=pl.DeviceIdType.MESH,
            )
        pl.semaphore_wait(barrier_sem, 2)

        rdma_to_right = pltpu.make_async_remote_copy(
            src_ref=x_ref.at[pl.ds(m_per - 1, 1), :],
            dst_ref=halo_buf.at[0],
            send_sem=send_sems.at[0],
            recv_sem=recv_sems.at[0],
            device_id=(right,),
            device_id_type=pl.DeviceIdType.MESH,
        )
        rdma_to_left = pltpu.make_async_remote_copy(
            src_ref=x_ref.at[pl.ds(0, 1), :],
            dst_ref=halo_buf.at[1],
            send_sem=send_sems.at[1],
            recv_sem=recv_sems.at[1],
            device_id=(left,),
            device_id_type=pl.DeviceIdType.MESH,
        )
        rdma_to_right.start()
        rdma_to_left.start()

        xv = x_ref[:, :]
        out_ref[1 : m_per - 1, :] = (
            0.25 * xv[0 : m_per - 2, :]
            + 0.5 * xv[1 : m_per - 1, :]
            + 0.25 * xv[2:m_per, :]
        )
        top_partial = jnp.where(
            my == 0, xv[0:1, :], 0.5 * xv[0:1, :] + 0.25 * xv[1:2, :]
        )
        bot_partial = jnp.where(
            my == N_DEV - 1,
            xv[m_per - 1 : m_per, :],
            0.25 * xv[m_per - 2 : m_per - 1, :] + 0.5 * xv[m_per - 1 : m_per, :],
        )
        halo_scale = jnp.where(my == 0, 0.0, 0.25)
        halo_scale_bot = jnp.where(my == N_DEV - 1, 0.0, 0.25)

        rdma_to_right.wait()
        out_ref[0:1, :] = top_partial + halo_scale * halo_buf[0, :, :]

        rdma_to_left.wait()
        out_ref[m_per - 1 : m_per, :] = (
            bot_partial + halo_scale_bot * halo_buf[1, :, :]
        )

    return pl.pallas_call(
        body,
        out_shape=jax.ShapeDtypeStruct((m_per, n), x.dtype),
        in_specs=[pl.BlockSpec(memory_space=pltpu.VMEM)],
        out_specs=pl.BlockSpec(memory_space=pltpu.VMEM),
        scratch_shapes=[
            pltpu.VMEM((2, 1, n), x.dtype),
            pltpu.SemaphoreType.DMA((2,)),
            pltpu.SemaphoreType.DMA((2,)),
        ],
        compiler_params=pltpu.CompilerParams(collective_id=0),
    )(x)
